# baseline (speedup 1.0000x reference)
import jax
import jax.numpy as jnp
from jax.experimental import pallas as pl

_NUM_USERS = 1_000_000
_NUM_SUBGROUPS = 1_000
_NUM_CHOICES = 100_000
_D = 64

_UW = 512
_UROWS = _NUM_USERS * _D // _UW
_CW = 128
_CROWS = _NUM_CHOICES * _D // _CW

_GRID = 25
_UB = _UROWS // _GRID
_CB = _CROWS // _GRID
_SB = _NUM_SUBGROUPS // _GRID


def _tables_kernel(sub_ref, cho_ref, user_ref, sub_out, cho_out, user_out):
    sub_out[...] = sub_ref[...]
    cho_out[...] = cho_ref[...]
    user_out[...] = jnp.broadcast_to(user_ref[...], (_UB, _UW))


def kernel(sub_w, cho_w, user_token):
    cho_r = cho_w.reshape(_CROWS, _CW)
    user_row = jnp.tile(user_token, (1, _UW // _D))

    sub_o, cho_o, user_o = pl.pallas_call(
        _tables_kernel,
        grid=(_GRID,),
        in_specs=[
            pl.BlockSpec((_SB, _D), lambda i: (i, 0)),
            pl.BlockSpec((_CB, _CW), lambda i: (i, 0)),
            pl.BlockSpec((1, _UW), lambda i: (0, 0)),
        ],
        out_specs=[
            pl.BlockSpec((_SB, _D), lambda i: (i, 0)),
            pl.BlockSpec((_CB, _CW), lambda i: (i, 0)),
            pl.BlockSpec((_UB, _UW), lambda i: (i, 0)),
        ],
        out_shape=[
            jax.ShapeDtypeStruct((_NUM_SUBGROUPS, _D), jnp.float32),
            jax.ShapeDtypeStruct((_CROWS, _CW), jnp.float32),
            jax.ShapeDtypeStruct((_UROWS, _UW), jnp.float32),
        ],
    )(sub_w, cho_r, user_row)

    return (
        sub_o,
        cho_o.reshape(_NUM_CHOICES, _D),
        user_o.reshape(_NUM_USERS, _D),
    )


# speedup vs baseline: 1.3642x; 1.3642x over previous
import jax
import jax.numpy as jnp
from jax.experimental import pallas as pl
from jax.experimental.pallas import tpu as pltpu

_NUM_USERS = 1_000_000
_NUM_SUBGROUPS = 1_000
_NUM_CHOICES = 100_000
_D = 64

_SCR_ROWS = 8_000
_N_USER = _NUM_USERS // _SCR_ROWS
_N_CHO = 16
_CHO_ROWS = _NUM_CHOICES // _N_CHO


def _tables_kernel(user_ref, sub_hbm, cho_hbm,
                   sub_out, cho_out, user_out,
                   scratch, cho_vmem, sub_vmem,
                   sem_user, sem_cho_rd, sem_cho_wr, sem_sub):
    cho_reads = []
    for j in range(_N_CHO):
        c = pltpu.make_async_copy(
            cho_hbm.at[pl.ds(j * _CHO_ROWS, _CHO_ROWS), :],
            cho_vmem.at[pl.ds(j * _CHO_ROWS, _CHO_ROWS), :],
            sem_cho_rd.at[j])
        c.start()
        cho_reads.append(c)
    sub_read = pltpu.make_async_copy(sub_hbm, sub_vmem, sem_sub)
    sub_read.start()

    scratch[...] = jnp.broadcast_to(user_ref[...], (_SCR_ROWS, _D))

    user_copies = []
    for i in range(_N_USER):
        c = pltpu.make_async_copy(
            scratch,
            user_out.at[pl.ds(i * _SCR_ROWS, _SCR_ROWS), :],
            sem_user)
        c.start()
        user_copies.append(c)

    cho_writes = []
    for j in range(_N_CHO):
        cho_reads[j].wait()
        c = pltpu.make_async_copy(
            cho_vmem.at[pl.ds(j * _CHO_ROWS, _CHO_ROWS), :],
            cho_out.at[pl.ds(j * _CHO_ROWS, _CHO_ROWS), :],
            sem_cho_wr)
        c.start()
        cho_writes.append(c)

    sub_read.wait()
    sub_write = pltpu.make_async_copy(sub_vmem, sub_out, sem_sub)
    sub_write.start()

    for c in user_copies:
        c.wait()
    for c in cho_writes:
        c.wait()
    sub_write.wait()


def kernel(sub_w, cho_w, user_token):
    sub_o, cho_o, user_o = pl.pallas_call(
        _tables_kernel,
        in_specs=[
            pl.BlockSpec(memory_space=pltpu.MemorySpace.VMEM),
            pl.BlockSpec(memory_space=pltpu.MemorySpace.HBM),
            pl.BlockSpec(memory_space=pltpu.MemorySpace.HBM),
        ],
        out_specs=[
            pl.BlockSpec(memory_space=pltpu.MemorySpace.HBM),
            pl.BlockSpec(memory_space=pltpu.MemorySpace.HBM),
            pl.BlockSpec(memory_space=pltpu.MemorySpace.HBM),
        ],
        out_shape=[
            jax.ShapeDtypeStruct((_NUM_SUBGROUPS, _D), jnp.float32),
            jax.ShapeDtypeStruct((_NUM_CHOICES, _D), jnp.float32),
            jax.ShapeDtypeStruct((_NUM_USERS, _D), jnp.float32),
        ],
        scratch_shapes=[
            pltpu.VMEM((_SCR_ROWS, _D), jnp.float32),
            pltpu.VMEM((_NUM_CHOICES, _D), jnp.float32),
            pltpu.VMEM((_NUM_SUBGROUPS, _D), jnp.float32),
            pltpu.SemaphoreType.DMA,
            pltpu.SemaphoreType.DMA((_N_CHO,)),
            pltpu.SemaphoreType.DMA,
            pltpu.SemaphoreType.DMA,
        ],
    )(user_token, sub_w, cho_w)
    return (sub_o, cho_o, user_o)
